# baseline (speedup 1.0000x reference)
import functools

import jax
import jax.numpy as jnp
from jax import lax
from jax.experimental import pallas as pl
from jax.experimental.pallas import tpu as pltpu
from jax.experimental.pallas import tpu_sc as plsc

_B = 16384
_D = 64
_R = 64
_NREL = 32
_L2_LAMBDA = 1e-05

_NC = 2
_NS = 16
_NW = _NC * _NS
_TOT = 3 * _B
_BPW = _TOT // _NW
_PD = 2 * _D
_C = 128
_NCHUNK = _BPW // _C

_BLK = 2048
_NBLK = _B // _BLK

_QOFF = 276480
_CB = 4608
_NCB = _QOFF // _CB
_LASTB = 1100000 // _CB


def _pack2(lo, hi):
    lo16 = lax.bitcast_convert_type(lo.astype(jnp.bfloat16), jnp.uint16)
    hi16 = lax.bitcast_convert_type(hi.astype(jnp.bfloat16), jnp.uint16)
    u = lo16.astype(jnp.uint32) | (hi16.astype(jnp.uint32) << 16)
    return lax.bitcast_convert_type(u, jnp.float32)


def _tc_pair_body(a_ref, b_ref, c_ref, d_ref, out_ref):
    t0 = jnp.transpose(a_ref[...], (1, 0))
    t1 = jnp.transpose(b_ref[...], (1, 0))
    t2 = jnp.transpose(c_ref[...], (1, 0))
    t3 = jnp.transpose(d_ref[...], (1, 0))
    out_ref[...] = jnp.concatenate([_pack2(t0, t2), _pack2(t1, t3)], axis=1)


def _build_pair_table(entity_user_embed):
    tableT = jnp.swapaxes(entity_user_embed, 0, 1)
    return pl.pallas_call(
        _tc_pair_body,
        grid=(_NCB,),
        in_specs=[
            pl.BlockSpec((_D, _CB), lambda i: (0, i)),
            pl.BlockSpec((_D, _CB),
                         lambda i: (0, jnp.minimum(i + _NCB, _LASTB))),
            pl.BlockSpec((_D, _CB),
                         lambda i: (0, jnp.minimum(i + 2 * _NCB, _LASTB))),
            pl.BlockSpec((_D, _CB),
                         lambda i: (0, jnp.minimum(i + 3 * _NCB, _LASTB))),
        ],
        out_specs=pl.BlockSpec((_CB, _PD), lambda i: (i, 0)),
        out_shape=jax.ShapeDtypeStruct((_QOFF, _PD), jnp.float32),
    )(tableT, tableT, tableT, tableT)


def _sc_gather_body(table_hbm, sid_hbm, out_hbm, sid_v, pair0, pair1,
                    pair2, pair3, sem0, sem1, sem2, sem3):
    wid = lax.axis_index("s") * _NC + lax.axis_index("c")
    base = wid * _BPW
    pltpu.sync_copy(sid_hbm.at[pl.ds(base, _BPW)], sid_v)
    bufs = (pair0, pair1, pair2, pair3)
    sems = (sem0, sem1, sem2, sem3)
    nbuf = 4

    def fire(g):
        return pltpu.async_copy(
            table_hbm.at[sid_v.at[pl.ds(g * _C, _C)]], bufs[g % nbuf],
            sems[g % nbuf])

    pend = [fire(g) for g in range(nbuf - 1)]
    for g in range(_NCHUNK):
        if g + nbuf - 1 < _NCHUNK:
            pend.append(fire(g + nbuf - 1))
        pend.pop(0).wait()
        pltpu.sync_copy(bufs[g % nbuf], out_hbm.at[pl.ds(base + g * _C, _C)])


@functools.cache
def _sc_gather():
    return pl.kernel(
        _sc_gather_body,
        out_type=jax.ShapeDtypeStruct((_TOT, _PD), jnp.float32),
        mesh=plsc.VectorSubcoreMesh(core_axis_name="c", subcore_axis_name="s",
                                    num_cores=_NC, num_subcores=_NS),
        scratch_types=[
            pltpu.VMEM((_BPW,), jnp.int32),
            pltpu.VMEM((_C, _PD), jnp.float32),
            pltpu.VMEM((_C, _PD), jnp.float32),
            pltpu.VMEM((_C, _PD), jnp.float32),
            pltpu.VMEM((_C, _PD), jnp.float32),
            pltpu.SemaphoreType.DMA,
            pltpu.SemaphoreType.DMA,
            pltpu.SemaphoreType.DMA,
            pltpu.SemaphoreType.DMA,
        ],
    )


def _tc_loss_body(r_ref, ph_ref, pp_ref, pn_ref, xh_ref, xp_ref, xn_ref,
                  mflat_ref, rel_ref, out_ref):
    i = pl.program_id(0)
    r = r_ref[0, 0, :]

    col32 = lax.broadcasted_iota(jnp.int32, (_BLK, _NREL), 1)
    onehot = (r[:, None] == col32).astype(jnp.bfloat16)
    r_emb = jnp.dot(onehot, rel_ref[...].astype(jnp.bfloat16),
                    preferred_element_type=jnp.float32)

    colk = lax.broadcasted_iota(jnp.int32, (_BLK, _NREL * _D), 1) // _D
    m16 = (r[:, None] == colk).astype(jnp.bfloat16)
    mflat = mflat_ref[...]

    def proj(x_ref, p_ref):
        x2 = x_ref[...]
        sel = p_ref[0, 0, 0, :]
        xg = jnp.where((sel & 1)[:, None] == 1, x2[:, _D:], x2[:, :_D])
        u = lax.bitcast_convert_type(xg, jnp.uint32)
        u16 = jnp.where((sel >> 1)[:, None] == 1, u >> 16, u & 0xFFFF)
        x16 = lax.bitcast_convert_type(u16.astype(jnp.uint16), jnp.bfloat16)
        tiled = jnp.concatenate([x16] * _NREL, axis=1)
        return jnp.dot(tiled * m16, mflat,
                       preferred_element_type=jnp.float32)

    mh = proj(xh_ref, ph_ref)
    mp = proj(xp_ref, pp_ref)
    mn = proj(xn_ref, pn_ref)

    pos = jnp.sum((mh + r_emb - mp) ** 2, axis=1, keepdims=True)
    neg = jnp.sum((mh + r_emb - mn) ** 2, axis=1, keepdims=True)
    d = neg - pos
    kg = jnp.sum(jnp.maximum(-d, 0.0) + jnp.log(1.0 + jnp.exp(-jnp.abs(d))))
    l2 = 0.5 * (jnp.sum(mh * mh) + jnp.sum(r_emb * r_emb)
                + jnp.sum(mp * mp) + jnp.sum(mn * mn))
    part = jnp.reshape((kg + _L2_LAMBDA * l2) * (1.0 / _B), (1, 1))

    @pl.when(i == 0)
    def _():
        out_ref[...] = jnp.zeros((1, 1), jnp.float32)

    out_ref[...] += part


def kernel(h, r, pos_t, neg_t, entity_user_embed, relation_embed, trans_M):
    idx = jnp.concatenate([h, pos_t, neg_t]).astype(jnp.int32)
    sel = idx // _QOFF
    sid = idx - _QOFF * sel
    par3 = sel.reshape(3, _NBLK, 1, _BLK)
    table2 = _build_pair_table(entity_user_embed)
    gathered = _sc_gather()(table2, sid)

    r3 = r.astype(jnp.int32).reshape(_NBLK, 1, _BLK)
    mflat = trans_M.reshape(_NREL * _D, _R).astype(jnp.bfloat16)

    out = pl.pallas_call(
        _tc_loss_body,
        grid=(_NBLK,),
        in_specs=[
            pl.BlockSpec((1, 1, _BLK), lambda i: (i, 0, 0)),
            pl.BlockSpec((1, 1, 1, _BLK), lambda i: (0, i, 0, 0)),
            pl.BlockSpec((1, 1, 1, _BLK), lambda i: (1, i, 0, 0)),
            pl.BlockSpec((1, 1, 1, _BLK), lambda i: (2, i, 0, 0)),
            pl.BlockSpec((_BLK, _PD), lambda i: (i, 0)),
            pl.BlockSpec((_BLK, _PD), lambda i: (i + _NBLK, 0)),
            pl.BlockSpec((_BLK, _PD), lambda i: (i + 2 * _NBLK, 0)),
            pl.BlockSpec((_NREL * _D, _R), lambda i: (0, 0)),
            pl.BlockSpec((_NREL, _R), lambda i: (0, 0)),
        ],
        out_specs=pl.BlockSpec((1, 1), lambda i: (0, 0)),
        out_shape=jax.ShapeDtypeStruct((1, 1), jnp.float32),
    )(r3, par3, par3, par3, gathered, gathered, gathered, mflat,
      relation_embed)
    return out[0, 0]


# speedup vs baseline: 1.0754x; 1.0754x over previous
import functools

import jax
import jax.numpy as jnp
from jax import lax
from jax.experimental import pallas as pl
from jax.experimental.pallas import tpu as pltpu
from jax.experimental.pallas import tpu_sc as plsc

_B = 16384
_D = 64
_R = 64
_NREL = 32
_L2_LAMBDA = 1e-05

_NC = 2
_NS = 16
_NW = _NC * _NS
_TOT = 3 * _B
_BPW = _TOT // _NW
_PD = 2 * _D
_C = 128
_NCHUNK = _BPW // _C

_BLK = 2048
_NBLK = _B // _BLK

_QOFF = 276480
_CB = 13824
_NCB = _QOFF // _CB
_LASTB = 1100000 // _CB


def _pack2(lo, hi):
    lo16 = lax.bitcast_convert_type(lo.astype(jnp.bfloat16), jnp.uint16)
    hi16 = lax.bitcast_convert_type(hi.astype(jnp.bfloat16), jnp.uint16)
    u = lo16.astype(jnp.uint32) | (hi16.astype(jnp.uint32) << 16)
    return lax.bitcast_convert_type(u, jnp.float32)


def _tc_pair_body(a_ref, b_ref, c_ref, d_ref, out_ref):
    t0 = jnp.transpose(a_ref[...], (1, 0))
    t1 = jnp.transpose(b_ref[...], (1, 0))
    t2 = jnp.transpose(c_ref[...], (1, 0))
    t3 = jnp.transpose(d_ref[...], (1, 0))
    out_ref[...] = jnp.concatenate([_pack2(t0, t2), _pack2(t1, t3)], axis=1)


def _build_pair_table(entity_user_embed):
    tableT = jnp.swapaxes(entity_user_embed, 0, 1)
    return pl.pallas_call(
        _tc_pair_body,
        grid=(_NCB,),
        in_specs=[
            pl.BlockSpec((_D, _CB), lambda i: (0, i)),
            pl.BlockSpec((_D, _CB),
                         lambda i: (0, jnp.minimum(i + _NCB, _LASTB))),
            pl.BlockSpec((_D, _CB),
                         lambda i: (0, jnp.minimum(i + 2 * _NCB, _LASTB))),
            pl.BlockSpec((_D, _CB),
                         lambda i: (0, jnp.minimum(i + 3 * _NCB, _LASTB))),
        ],
        out_specs=pl.BlockSpec((_CB, _PD), lambda i: (i, 0)),
        out_shape=jax.ShapeDtypeStruct((_QOFF, _PD), jnp.float32),
    )(tableT, tableT, tableT, tableT)


def _sc_gather_body(table_hbm, sid_hbm, out_hbm, sid_v, pair0, pair1,
                    pair2, pair3, sem0, sem1, sem2, sem3):
    wid = lax.axis_index("s") * _NC + lax.axis_index("c")
    base = wid * _BPW
    pltpu.sync_copy(sid_hbm.at[pl.ds(base, _BPW)], sid_v)
    bufs = (pair0, pair1, pair2, pair3)
    sems = (sem0, sem1, sem2, sem3)
    nbuf = 4

    def fire(g):
        return pltpu.async_copy(
            table_hbm.at[sid_v.at[pl.ds(g * _C, _C)]], bufs[g % nbuf],
            sems[g % nbuf])

    pend = [fire(g) for g in range(nbuf - 1)]
    for g in range(_NCHUNK):
        if g + nbuf - 1 < _NCHUNK:
            pend.append(fire(g + nbuf - 1))
        pend.pop(0).wait()
        pltpu.sync_copy(bufs[g % nbuf], out_hbm.at[pl.ds(base + g * _C, _C)])


@functools.cache
def _sc_gather():
    return pl.kernel(
        _sc_gather_body,
        out_type=jax.ShapeDtypeStruct((_TOT, _PD), jnp.float32),
        mesh=plsc.VectorSubcoreMesh(core_axis_name="c", subcore_axis_name="s",
                                    num_cores=_NC, num_subcores=_NS),
        scratch_types=[
            pltpu.VMEM((_BPW,), jnp.int32),
            pltpu.VMEM((_C, _PD), jnp.float32),
            pltpu.VMEM((_C, _PD), jnp.float32),
            pltpu.VMEM((_C, _PD), jnp.float32),
            pltpu.VMEM((_C, _PD), jnp.float32),
            pltpu.SemaphoreType.DMA,
            pltpu.SemaphoreType.DMA,
            pltpu.SemaphoreType.DMA,
            pltpu.SemaphoreType.DMA,
        ],
    )


def _tc_loss_body(r_ref, ph_ref, pp_ref, pn_ref, xh_ref, xp_ref, xn_ref,
                  mflat_ref, rel_ref, out_ref):
    i = pl.program_id(0)
    r = r_ref[0, 0, :]

    col32 = lax.broadcasted_iota(jnp.int32, (_BLK, _NREL), 1)
    onehot = (r[:, None] == col32).astype(jnp.bfloat16)
    r_emb = jnp.dot(onehot, rel_ref[...].astype(jnp.bfloat16),
                    preferred_element_type=jnp.float32)

    colk = lax.broadcasted_iota(jnp.int32, (_BLK, _NREL * _D), 1) // _D
    m16 = (r[:, None] == colk).astype(jnp.bfloat16)
    mflat = mflat_ref[...]

    def proj(x_ref, p_ref):
        x2 = x_ref[...]
        sel = p_ref[0, 0, 0, :]
        xg = jnp.where((sel & 1)[:, None] == 1, x2[:, _D:], x2[:, :_D])
        u = lax.bitcast_convert_type(xg, jnp.uint32)
        u16 = jnp.where((sel >> 1)[:, None] == 1, u >> 16, u & 0xFFFF)
        x16 = lax.bitcast_convert_type(u16.astype(jnp.uint16), jnp.bfloat16)
        tiled = jnp.concatenate([x16] * _NREL, axis=1)
        return jnp.dot(tiled * m16, mflat,
                       preferred_element_type=jnp.float32)

    mh = proj(xh_ref, ph_ref)
    mp = proj(xp_ref, pp_ref)
    mn = proj(xn_ref, pn_ref)

    pos = jnp.sum((mh + r_emb - mp) ** 2, axis=1, keepdims=True)
    neg = jnp.sum((mh + r_emb - mn) ** 2, axis=1, keepdims=True)
    d = neg - pos
    kg = jnp.sum(jnp.maximum(-d, 0.0) + jnp.log(1.0 + jnp.exp(-jnp.abs(d))))
    l2 = 0.5 * (jnp.sum(mh * mh) + jnp.sum(r_emb * r_emb)
                + jnp.sum(mp * mp) + jnp.sum(mn * mn))
    part = jnp.reshape((kg + _L2_LAMBDA * l2) * (1.0 / _B), (1, 1))

    @pl.when(i == 0)
    def _():
        out_ref[...] = jnp.zeros((1, 1), jnp.float32)

    out_ref[...] += part


def kernel(h, r, pos_t, neg_t, entity_user_embed, relation_embed, trans_M):
    idx = jnp.concatenate([h, pos_t, neg_t]).astype(jnp.int32)
    sel = idx // _QOFF
    sid = idx - _QOFF * sel
    par3 = sel.reshape(3, _NBLK, 1, _BLK)
    table2 = _build_pair_table(entity_user_embed)
    gathered = _sc_gather()(table2, sid)

    r3 = r.astype(jnp.int32).reshape(_NBLK, 1, _BLK)
    mflat = trans_M.reshape(_NREL * _D, _R).astype(jnp.bfloat16)

    out = pl.pallas_call(
        _tc_loss_body,
        grid=(_NBLK,),
        in_specs=[
            pl.BlockSpec((1, 1, _BLK), lambda i: (i, 0, 0)),
            pl.BlockSpec((1, 1, 1, _BLK), lambda i: (0, i, 0, 0)),
            pl.BlockSpec((1, 1, 1, _BLK), lambda i: (1, i, 0, 0)),
            pl.BlockSpec((1, 1, 1, _BLK), lambda i: (2, i, 0, 0)),
            pl.BlockSpec((_BLK, _PD), lambda i: (i, 0)),
            pl.BlockSpec((_BLK, _PD), lambda i: (i + _NBLK, 0)),
            pl.BlockSpec((_BLK, _PD), lambda i: (i + 2 * _NBLK, 0)),
            pl.BlockSpec((_NREL * _D, _R), lambda i: (0, 0)),
            pl.BlockSpec((_NREL, _R), lambda i: (0, 0)),
        ],
        out_specs=pl.BlockSpec((1, 1), lambda i: (0, 0)),
        out_shape=jax.ShapeDtypeStruct((1, 1), jnp.float32),
    )(r3, par3, par3, par3, gathered, gathered, gathered, mflat,
      relation_embed)
    return out[0, 0]


# speedup vs baseline: 1.0789x; 1.0032x over previous
import functools

import jax
import jax.numpy as jnp
from jax import lax
from jax.experimental import pallas as pl
from jax.experimental.pallas import tpu as pltpu
from jax.experimental.pallas import tpu_sc as plsc

_B = 16384
_D = 64
_R = 64
_NREL = 32
_L2_LAMBDA = 1e-05

_NC = 2
_NS = 16
_NW = _NC * _NS
_TOT = 3 * _B
_BPW = _TOT // _NW
_PD = 2 * _D
_C = 128
_NCHUNK = _BPW // _C

_BLK = 4096
_NBLK = _B // _BLK

_QOFF = 276480
_CB = 13824
_NCB = _QOFF // _CB
_LASTB = 1100000 // _CB


def _pack2(lo, hi):
    lo16 = lax.bitcast_convert_type(lo.astype(jnp.bfloat16), jnp.uint16)
    hi16 = lax.bitcast_convert_type(hi.astype(jnp.bfloat16), jnp.uint16)
    u = lo16.astype(jnp.uint32) | (hi16.astype(jnp.uint32) << 16)
    return lax.bitcast_convert_type(u, jnp.float32)


def _tc_pair_body(a_ref, b_ref, c_ref, d_ref, out_ref):
    t0 = jnp.transpose(a_ref[...], (1, 0))
    t1 = jnp.transpose(b_ref[...], (1, 0))
    t2 = jnp.transpose(c_ref[...], (1, 0))
    t3 = jnp.transpose(d_ref[...], (1, 0))
    out_ref[...] = jnp.concatenate([_pack2(t0, t2), _pack2(t1, t3)], axis=1)


def _build_pair_table(entity_user_embed):
    tableT = jnp.swapaxes(entity_user_embed, 0, 1)
    return pl.pallas_call(
        _tc_pair_body,
        grid=(_NCB,),
        in_specs=[
            pl.BlockSpec((_D, _CB), lambda i: (0, i)),
            pl.BlockSpec((_D, _CB),
                         lambda i: (0, jnp.minimum(i + _NCB, _LASTB))),
            pl.BlockSpec((_D, _CB),
                         lambda i: (0, jnp.minimum(i + 2 * _NCB, _LASTB))),
            pl.BlockSpec((_D, _CB),
                         lambda i: (0, jnp.minimum(i + 3 * _NCB, _LASTB))),
        ],
        out_specs=pl.BlockSpec((_CB, _PD), lambda i: (i, 0)),
        out_shape=jax.ShapeDtypeStruct((_QOFF, _PD), jnp.float32),
    )(tableT, tableT, tableT, tableT)


def _sc_gather_body(table_hbm, sid_hbm, out_hbm, sid_v, pair0, pair1,
                    pair2, pair3, sem0, sem1, sem2, sem3):
    wid = lax.axis_index("s") * _NC + lax.axis_index("c")
    base = wid * _BPW
    pltpu.sync_copy(sid_hbm.at[pl.ds(base, _BPW)], sid_v)
    bufs = (pair0, pair1, pair2, pair3)
    sems = (sem0, sem1, sem2, sem3)
    nbuf = 4

    def fire(g):
        return pltpu.async_copy(
            table_hbm.at[sid_v.at[pl.ds(g * _C, _C)]], bufs[g % nbuf],
            sems[g % nbuf])

    pend = [fire(g) for g in range(nbuf - 1)]
    for g in range(_NCHUNK):
        if g + nbuf - 1 < _NCHUNK:
            pend.append(fire(g + nbuf - 1))
        pend.pop(0).wait()
        pltpu.sync_copy(bufs[g % nbuf], out_hbm.at[pl.ds(base + g * _C, _C)])


@functools.cache
def _sc_gather():
    return pl.kernel(
        _sc_gather_body,
        out_type=jax.ShapeDtypeStruct((_TOT, _PD), jnp.float32),
        mesh=plsc.VectorSubcoreMesh(core_axis_name="c", subcore_axis_name="s",
                                    num_cores=_NC, num_subcores=_NS),
        scratch_types=[
            pltpu.VMEM((_BPW,), jnp.int32),
            pltpu.VMEM((_C, _PD), jnp.float32),
            pltpu.VMEM((_C, _PD), jnp.float32),
            pltpu.VMEM((_C, _PD), jnp.float32),
            pltpu.VMEM((_C, _PD), jnp.float32),
            pltpu.SemaphoreType.DMA,
            pltpu.SemaphoreType.DMA,
            pltpu.SemaphoreType.DMA,
            pltpu.SemaphoreType.DMA,
        ],
    )


def _tc_loss_body(r_ref, ph_ref, pp_ref, pn_ref, xh_ref, xp_ref, xn_ref,
                  mflat_ref, rel_ref, out_ref):
    i = pl.program_id(0)
    r = r_ref[0, 0, :]

    col32 = lax.broadcasted_iota(jnp.int32, (_BLK, _NREL), 1)
    onehot = (r[:, None] == col32).astype(jnp.bfloat16)
    r_emb = jnp.dot(onehot, rel_ref[...].astype(jnp.bfloat16),
                    preferred_element_type=jnp.float32)

    colk = lax.broadcasted_iota(jnp.int32, (_BLK, _NREL * _D), 1) // _D
    m16 = (r[:, None] == colk).astype(jnp.bfloat16)
    mflat = mflat_ref[...]

    def proj(x_ref, p_ref):
        x2 = x_ref[...]
        sel = p_ref[0, 0, 0, :]
        xg = jnp.where((sel & 1)[:, None] == 1, x2[:, _D:], x2[:, :_D])
        u = lax.bitcast_convert_type(xg, jnp.uint32)
        u16 = jnp.where((sel >> 1)[:, None] == 1, u >> 16, u & 0xFFFF)
        x16 = lax.bitcast_convert_type(u16.astype(jnp.uint16), jnp.bfloat16)
        tiled = jnp.concatenate([x16] * _NREL, axis=1)
        return jnp.dot(tiled * m16, mflat,
                       preferred_element_type=jnp.float32)

    mh = proj(xh_ref, ph_ref)
    mp = proj(xp_ref, pp_ref)
    mn = proj(xn_ref, pn_ref)

    pos = jnp.sum((mh + r_emb - mp) ** 2, axis=1, keepdims=True)
    neg = jnp.sum((mh + r_emb - mn) ** 2, axis=1, keepdims=True)
    d = neg - pos
    kg = jnp.sum(jnp.maximum(-d, 0.0) + jnp.log(1.0 + jnp.exp(-jnp.abs(d))))
    l2 = 0.5 * (jnp.sum(mh * mh) + jnp.sum(r_emb * r_emb)
                + jnp.sum(mp * mp) + jnp.sum(mn * mn))
    part = jnp.reshape((kg + _L2_LAMBDA * l2) * (1.0 / _B), (1, 1))

    @pl.when(i == 0)
    def _():
        out_ref[...] = jnp.zeros((1, 1), jnp.float32)

    out_ref[...] += part


def kernel(h, r, pos_t, neg_t, entity_user_embed, relation_embed, trans_M):
    idx = jnp.concatenate([h, pos_t, neg_t]).astype(jnp.int32)
    sel = idx // _QOFF
    sid = idx - _QOFF * sel
    par3 = sel.reshape(3, _NBLK, 1, _BLK)
    table2 = _build_pair_table(entity_user_embed)
    gathered = _sc_gather()(table2, sid)

    r3 = r.astype(jnp.int32).reshape(_NBLK, 1, _BLK)
    mflat = trans_M.reshape(_NREL * _D, _R).astype(jnp.bfloat16)

    out = pl.pallas_call(
        _tc_loss_body,
        grid=(_NBLK,),
        in_specs=[
            pl.BlockSpec((1, 1, _BLK), lambda i: (i, 0, 0)),
            pl.BlockSpec((1, 1, 1, _BLK), lambda i: (0, i, 0, 0)),
            pl.BlockSpec((1, 1, 1, _BLK), lambda i: (1, i, 0, 0)),
            pl.BlockSpec((1, 1, 1, _BLK), lambda i: (2, i, 0, 0)),
            pl.BlockSpec((_BLK, _PD), lambda i: (i, 0)),
            pl.BlockSpec((_BLK, _PD), lambda i: (i + _NBLK, 0)),
            pl.BlockSpec((_BLK, _PD), lambda i: (i + 2 * _NBLK, 0)),
            pl.BlockSpec((_NREL * _D, _R), lambda i: (0, 0)),
            pl.BlockSpec((_NREL, _R), lambda i: (0, 0)),
        ],
        out_specs=pl.BlockSpec((1, 1), lambda i: (0, 0)),
        out_shape=jax.ShapeDtypeStruct((1, 1), jnp.float32),
    )(r3, par3, par3, par3, gathered, gathered, gathered, mflat,
      relation_embed)
    return out[0, 0]
